# Initial kernel scaffold; baseline (speedup 1.0000x reference)
import functools

import jax
import jax.numpy as jnp
from jax import lax
from jax.experimental import pallas as pl
from jax.experimental.pallas import tpu as pltpu
from jax.experimental.pallas import tpu_sc as plsc

_NUM_CORES = 2
_NUM_SUBCORES = 16
_NW = _NUM_CORES * _NUM_SUBCORES

_B = 4096
_L = 200
_D = 64
_REPEATS = 2
_N = _B * _L
_PER_TILE = _N // _NW

_CHUNK = 512
_GATHER = 128
_STEPS = _PER_TILE // _CHUNK
_NGATHER = _CHUNK // _GATHER


def _body(code_hbm, table_hbm, out_hbm, idx_v, rows_v, dup_v, sem):
    wid = lax.axis_index("s") * _NUM_CORES + lax.axis_index("c")
    tile_base = wid * _PER_TILE

    def step(i, _):
        base = tile_base + i * _CHUNK
        pltpu.sync_copy(code_hbm.at[pl.ds(base, _CHUNK)], idx_v)
        copies = []
        for g in range(_NGATHER):
            copies.append(pltpu.async_copy(
                table_hbm.at[idx_v.at[pl.ds(g * _GATHER, _GATHER)]],
                rows_v.at[pl.ds(g * _GATHER, _GATHER)],
                sem,
            ))
        for c in copies:
            c.wait()

        def dup(r, _):
            for d in range(_D // 16):
                v = rows_v[r, pl.ds(d * 16, 16)]
                dup_v[r, 0, pl.ds(d * 16, 16)] = v
                dup_v[r, 1, pl.ds(d * 16, 16)] = v
            return 0

        lax.fori_loop(0, _CHUNK, dup, 0, unroll=2)

        pltpu.sync_copy(
            dup_v,
            out_hbm.at[pl.ds(_REPEATS * base, _REPEATS * _CHUNK)],
        )
        return 0

    lax.fori_loop(0, _STEPS, step, 0)


@jax.jit
def _run(code_flat, table):
    k = pl.kernel(
        _body,
        out_type=jax.ShapeDtypeStruct((_REPEATS * _N, _D), jnp.float32),
        mesh=plsc.VectorSubcoreMesh(
            core_axis_name="c", subcore_axis_name="s",
            num_cores=_NUM_CORES, num_subcores=_NUM_SUBCORES,
        ),
        scratch_types=[
            pltpu.VMEM((_CHUNK,), jnp.int32),
            pltpu.VMEM((_CHUNK, _D), jnp.float32),
            pltpu.VMEM((_CHUNK, _REPEATS, _D), jnp.float32),
            pltpu.SemaphoreType.DMA,
        ],
    )
    return k(code_flat, table)


def kernel(code, table):
    code_flat = code.reshape(_N).astype(jnp.int32)
    out2 = _run(code_flat, table)
    return out2.reshape(_B, _L * _REPEATS, _D)


# speedup vs baseline: 1.9624x; 1.9624x over previous
import functools

import jax
import jax.numpy as jnp
from jax import lax
from jax.experimental import pallas as pl
from jax.experimental.pallas import tpu as pltpu
from jax.experimental.pallas import tpu_sc as plsc

_NUM_CORES = 2
_NUM_SUBCORES = 16
_NW = _NUM_CORES * _NUM_SUBCORES

_B = 4096
_L = 200
_D = 64
_REPEATS = 2
_N = _B * _L
_PER_TILE = _N // _NW

_CHUNK = 512
_GATHER = 128
_STEPS = _PER_TILE // _CHUNK
_NGATHER = _CHUNK // _GATHER


def _body(code_hbm, table_hbm, out_hbm, idx_v, rows_v, dup_v, sem):
    wid = lax.axis_index("s") * _NUM_CORES + lax.axis_index("c")
    tile_base = wid * _PER_TILE

    def step(i, _):
        base = tile_base + i * _CHUNK
        pltpu.sync_copy(code_hbm.at[pl.ds(base, _CHUNK)], idx_v)
        copies = []
        for g in range(_NGATHER):
            copies.append(pltpu.async_copy(
                table_hbm.at[idx_v.at[pl.ds(g * _GATHER, _GATHER)]],
                rows_v.at[pl.ds(g * _GATHER, _GATHER)],
                sem,
            ))
        for c in copies:
            c.wait()

        def dup(r, _):
            for d in range(_D // 16):
                v = rows_v[r, pl.ds(d * 16, 16)]
                dup_v[2 * r, pl.ds(d * 16, 16)] = v
                dup_v[2 * r + 1, pl.ds(d * 16, 16)] = v
            return 0

        lax.fori_loop(0, _CHUNK, dup, 0, unroll=2)

        pltpu.sync_copy(
            dup_v,
            out_hbm.at[pl.ds(_REPEATS * base, _REPEATS * _CHUNK)],
        )
        return 0

    lax.fori_loop(0, _STEPS, step, 0)


@jax.jit
def _run(code_flat, table):
    k = pl.kernel(
        _body,
        out_type=jax.ShapeDtypeStruct((_REPEATS * _N, _D), jnp.float32),
        mesh=plsc.VectorSubcoreMesh(
            core_axis_name="c", subcore_axis_name="s",
            num_cores=_NUM_CORES, num_subcores=_NUM_SUBCORES,
        ),
        scratch_types=[
            pltpu.VMEM((_CHUNK,), jnp.int32),
            pltpu.VMEM((_CHUNK, _D), jnp.float32),
            pltpu.VMEM((_CHUNK * _REPEATS, _D), jnp.float32),
            pltpu.SemaphoreType.DMA,
        ],
        compiler_params=pltpu.CompilerParams(use_tc_tiling_on_sc=False),
    )
    return k(code_flat, table)


def kernel(code, table):
    code_flat = code.reshape(_N).astype(jnp.int32)
    out2 = _run(code_flat, table)
    return out2.reshape(_B, _L * _REPEATS, _D)


# speedup vs baseline: 2.3055x; 1.1748x over previous
import jax
import jax.numpy as jnp
from jax import lax
from jax.experimental import pallas as pl
from jax.experimental.pallas import tpu as pltpu
from jax.experimental.pallas import tpu_sc as plsc

_NUM_CORES = 2
_NUM_SUBCORES = 16
_NW = _NUM_CORES * _NUM_SUBCORES

_B = 4096
_L = 200
_D = 64
_REPEATS = 2
_N = _B * _L
_PER_TILE = _N // _NW

_CHUNK = 256
_GATHER = 128
_STEPS = _PER_TILE // _CHUNK
_NGATHER = _CHUNK // _GATHER
_NBUF = 2


def _body(code_hbm, table_hbm, out_hbm, idx_v, rows_v, dup_v, gsems, wsems):
    wid = lax.axis_index("s") * _NUM_CORES + lax.axis_index("c")
    tile_base = wid * _PER_TILE

    pltpu.sync_copy(code_hbm.at[pl.ds(tile_base, _PER_TILE)], idx_v)

    def fire_gathers(step, b):
        for g in range(_NGATHER):
            pltpu.async_copy(
                table_hbm.at[idx_v.at[pl.ds(step * _CHUNK + g * _GATHER,
                                            _GATHER)]],
                rows_v.at[b].at[pl.ds(g * _GATHER, _GATHER)],
                gsems[b],
            )

    def wait_gathers(step, b):
        for g in range(_NGATHER):
            pltpu.make_async_copy(
                table_hbm.at[idx_v.at[pl.ds(step * _CHUNK + g * _GATHER,
                                            _GATHER)]],
                rows_v.at[b].at[pl.ds(g * _GATHER, _GATHER)],
                gsems[b],
            ).wait()

    def out_copy(step, b):
        return pltpu.make_async_copy(
            dup_v.at[b],
            out_hbm.at[pl.ds(_REPEATS * (tile_base + step * _CHUNK),
                             _REPEATS * _CHUNK)],
            wsems[b],
        )

    for b in range(_NBUF):
        fire_gathers(b, b)

    def outer(i, _):
        for b in range(_NBUF):
            step = i * _NBUF + b
            wait_gathers(step, b)

            @pl.when(step >= _NBUF)
            def _():
                out_copy(step - _NBUF, b).wait()

            def dup(r, _):
                for d in range(_D // 16):
                    v = rows_v[b, r, pl.ds(d * 16, 16)]
                    dup_v[b, 2 * r, pl.ds(d * 16, 16)] = v
                    dup_v[b, 2 * r + 1, pl.ds(d * 16, 16)] = v
                return 0

            lax.fori_loop(0, _CHUNK, dup, 0, unroll=4)

            out_copy(step, b).start()

            @pl.when(step + _NBUF < _STEPS)
            def _():
                fire_gathers(step + _NBUF, b)
        return 0

    lax.fori_loop(0, _STEPS // _NBUF, outer, 0)

    for b in range(_NBUF):
        out_copy(_STEPS - _NBUF + b, b).wait()


@jax.jit
def _run(code_flat, table):
    k = pl.kernel(
        _body,
        out_type=jax.ShapeDtypeStruct((_REPEATS * _N, _D), jnp.float32),
        mesh=plsc.VectorSubcoreMesh(
            core_axis_name="c", subcore_axis_name="s",
            num_cores=_NUM_CORES, num_subcores=_NUM_SUBCORES,
        ),
        scratch_types=[
            pltpu.VMEM((_PER_TILE,), jnp.int32),
            pltpu.VMEM((_NBUF, _CHUNK, _D), jnp.float32),
            pltpu.VMEM((_NBUF, _CHUNK * _REPEATS, _D), jnp.float32),
            [pltpu.SemaphoreType.DMA] * _NBUF,
            [pltpu.SemaphoreType.DMA] * _NBUF,
        ],
        compiler_params=pltpu.CompilerParams(use_tc_tiling_on_sc=False),
    )
    return k(code_flat, table)


def kernel(code, table):
    code_flat = code.reshape(_N).astype(jnp.int32)
    out2 = _run(code_flat, table)
    return out2.reshape(_B, _L * _REPEATS, _D)
